# baseline (speedup 1.0000x reference)
import functools

import jax
import jax.numpy as jnp
from jax import lax
from jax.experimental import pallas as pl
from jax.experimental.pallas import tpu as pltpu
from jax.experimental.pallas import tpu_sc as plsc

N_NODES = 50000
N_EDGES = 800000
N_GRAPHS = 512
VOCAB = 100000
EMB_DIM = 16
TOK_PER_NODE = 4
HIDDEN = 64
OUT_DIM = 1000

NACC = 50176
SLAB = NACC // 16
E_PAD = 819200
EW_PER_TILE = 400
X_PAD = 200704
DW_PER_TILE = 200
NPAIR = NACC // 2

_mesh = plsc.VectorSubcoreMesh(core_axis_name="c", subcore_axis_name="s")
_sc_params = pltpu.CompilerParams(use_tc_tiling_on_sc=False)


@functools.partial(
    pl.kernel,
    mesh=_mesh,
    compiler_params=_sc_params,
    out_type=[
        jax.ShapeDtypeStruct((X_PAD, EMB_DIM), jnp.float32),
        jax.ShapeDtypeStruct((NACC,), jnp.float32),
        jax.ShapeDtypeStruct((NACC,), jnp.float32),
    ],
    scratch_types=[
        pltpu.VMEM((896, EMB_DIM), jnp.float32),
        pltpu.VMEM((7, 128), jnp.int32),
        pltpu.VMEM((1024,), jnp.int32),
        pltpu.VMEM((1, 128), jnp.int32),
        pltpu.VMEM((128,), jnp.float32),
        pltpu.VMEM((SLAB,), jnp.float32),
        pltpu.VMEM_SHARED((NACC,), jnp.float32),
        pltpu.SemaphoreType.DMA,
    ],
)
def _sc_emb_deg(table, xp, ei, h0, deg0, deg1, rows, xbuf,
                dstb1, dstb2, ones, zbuf, deg_sh, gsem):
    c = lax.axis_index("c")
    s = lax.axis_index("s")
    w = s * 2 + c

    def zf(i, _):
        zbuf[pl.ds(i * 16, 16)] = jnp.zeros((16,), jnp.float32)
        return 0
    lax.fori_loop(0, SLAB // 16, zf, 0)
    for i in range(8):
        ones[pl.ds(i * 16, 16)] = jnp.ones((16,), jnp.float32)
    pltpu.sync_copy(zbuf, deg_sh.at[pl.ds(s * SLAB, SLAB)])
    plsc.subcore_barrier()

    def emb(g, _):
        pltpu.sync_copy(xp.at[pl.ds(w * 49 + g * 7, 7)], xbuf)
        cps = []
        for j in range(7):
            cps.append(pltpu.async_copy(
                table.at[xbuf.at[j]], rows.at[pl.ds(j * 128, 128)], gsem))
        for cp in cps:
            cp.wait()
        pltpu.sync_copy(rows, h0.at[pl.ds(w * 6272 + g * 896, 896)])
        return 0
    lax.fori_loop(0, 7, emb, 0)

    nwin_d = jnp.where(w < 31, 196, 174)
    dbase = w * 196 * 128

    def deg(g, _):
        pos = lax.rem(g, 8)

        @pl.when(pos == 0)
        def _():
            pltpu.sync_copy(ei.at[1, pl.ds(dbase + g * 128, 1024)], dstb1)
        for v in range(8):
            dstb2[0, pl.ds(v * 16, 16)] = dstb1[pl.ds(pos * 128 + v * 16, 16)]
        pltpu.sync_copy(ones, deg_sh.at[dstb2.at[0]], add=True)
        return 0
    lax.fori_loop(0, nwin_d, deg, 0)

    plsc.subcore_barrier()

    @pl.when(c == 0)
    def _():
        pltpu.sync_copy(deg_sh.at[pl.ds(s * SLAB, SLAB)],
                        deg0.at[pl.ds(s * SLAB, SLAB)])

    @pl.when(c == 1)
    def _():
        pltpu.sync_copy(deg_sh.at[pl.ds(s * SLAB, SLAB)],
                        deg1.at[pl.ds(s * SLAB, SLAB)])


_NG = EW_PER_TILE // 2
_NCHUNK = EW_PER_TILE // 8


@functools.partial(
    pl.kernel,
    mesh=_mesh,
    compiler_params=_sc_params,
    out_type=jax.ShapeDtypeStruct((NACC, 2, 32), jnp.float32),
    scratch_types=[
        pltpu.VMEM((2048,), jnp.int32),
        pltpu.VMEM((2048,), jnp.int32),
        pltpu.VMEM((2, 128), jnp.int32),
        pltpu.VMEM((2, 128), jnp.int32),
        pltpu.VMEM((2, 128, 32), jnp.float32),
        pltpu.VMEM((196, 32), jnp.float32),
        pltpu.VMEM_SHARED((NACC, 32), jnp.float32),
        pltpu.SemaphoreType.DMA,
        pltpu.SemaphoreType.DMA,
        pltpu.SemaphoreType.DMA,
    ],
)
def _sc_edge(hh, ei, acc3, srcb, dstb, idxg, dst2, rows, zb, acc_sh,
             gsem, ssem, isem):
    c = lax.axis_index("c")
    s = lax.axis_index("s")
    nwin = jnp.where(s < 15, 391, 385)
    ebase = s * 391 * 128

    def zf(r, _):
        zb[r, pl.ds(0, 16)] = jnp.zeros((16,), jnp.float32)
        zb[r, pl.ds(16, 16)] = jnp.zeros((16,), jnp.float32)
        return 0
    lax.fori_loop(0, 196, zf, 0)
    for k in range(16):
        pltpu.sync_copy(zb, acc_sh.at[pl.ds(s * SLAB + k * 196, 196)])
    plsc.subcore_barrier()

    def stage(bslot, boff):
        for v in range(8):
            sl = pl.ds(v * 16, 16)
            so = pl.ds(boff + v * 16, 16)
            idxg[bslot, sl] = srcb[so] * 2 + c
            dst2[bslot, sl] = dstb[so]

    pltpu.sync_copy(ei.at[0, pl.ds(ebase, 1024)], srcb.at[pl.ds(0, 1024)])
    pltpu.sync_copy(ei.at[1, pl.ds(ebase, 1024)], dstb.at[pl.ds(0, 1024)])
    pltpu.async_copy(ei.at[0, pl.ds(ebase + 1024, 1024)],
                     srcb.at[pl.ds(1024, 1024)], isem)
    pltpu.async_copy(ei.at[1, pl.ds(ebase + 1024, 1024)],
                     dstb.at[pl.ds(1024, 1024)], isem)
    stage(0, 0)
    pltpu.async_copy(hh.at[idxg.at[0]], rows.at[0], gsem)

    def step(g, _):
        slot = lax.rem(g, 2)
        pos = lax.rem(g, 8)
        k = g // 8
        kslot = lax.rem(k, 2)

        pltpu.make_async_copy(
            hh.at[idxg.at[slot]], rows.at[slot], gsem).wait()

        @pl.when((pos == 7) & (g < nwin - 1))
        def _():
            pltpu.make_async_copy(ei.at[0, pl.ds(ebase, 1024)],
                                  srcb.at[pl.ds(0, 1024)], isem).wait()
            pltpu.make_async_copy(ei.at[1, pl.ds(ebase, 1024)],
                                  dstb.at[pl.ds(0, 1024)], isem).wait()

        @pl.when(g < nwin - 1)
        def _():
            stage(1 - slot, lax.rem(g + 1, 16) * 128)
            pltpu.async_copy(hh.at[idxg.at[1 - slot]], rows.at[1 - slot],
                             gsem)

        pltpu.async_copy(rows.at[slot], acc_sh.at[dst2.at[slot]], ssem,
                         add=True).wait()

        @pl.when((pos == 7) & (k < 47))
        def _():
            off = ebase + (k + 2) * 1024
            pltpu.async_copy(ei.at[0, pl.ds(off, 1024)],
                             srcb.at[pl.ds(kslot * 1024, 1024)], isem)
            pltpu.async_copy(ei.at[1, pl.ds(off, 1024)],
                             dstb.at[pl.ds(kslot * 1024, 1024)], isem)
        return 0
    lax.fori_loop(0, nwin, step, 0)

    plsc.subcore_barrier()
    pltpu.sync_copy(acc_sh.at[pl.ds(s * SLAB, SLAB)],
                    acc3.at[pl.ds(s * SLAB, SLAB), c])


_BP = 512
_GRID = NPAIR // _BP


def _dinv_pair(d_ref, e_ref):
    d2 = lax.rsqrt(d_ref[...] + e_ref[...] + 1.0)
    return jnp.concatenate(
        [jnp.broadcast_to(d2[:, 0:1], (_BP, HIDDEN)),
         jnp.broadcast_to(d2[:, 1:2], (_BP, HIDDEN))], axis=1)


def _tc_k1(h0_ref, d_ref, e_ref, w_ref, o_ref):
    hw = jnp.dot(h0_ref[...], w_ref[...], preferred_element_type=jnp.float32)
    o_ref[...] = hw * _dinv_pair(d_ref, e_ref)


def _tc_k2(acc_ref, hwp_ref, d_ref, e_ref, w_ref, b_ref, o_ref):
    dinv = _dinv_pair(d_ref, e_ref)
    h1 = jnp.maximum((acc_ref[...] + hwp_ref[...]) * dinv + b_ref[...], 0.0)
    hw = jnp.dot(h1, w_ref[...], preferred_element_type=jnp.float32)
    o_ref[...] = hw * dinv


def _tc_k3(acc_ref, hwp_ref, d_ref, e_ref, b_ref, bat_ref, w3_ref,
           b3_ref, o_ref, pooled):
    i = pl.program_id(0)

    @pl.when(i == 0)
    def _():
        pooled[...] = jnp.zeros((N_GRAPHS, HIDDEN), jnp.float32)

    dinv = _dinv_pair(d_ref, e_ref)
    h2 = jnp.maximum((acc_ref[...] + hwp_ref[...]) * dinv + b_ref[...], 0.0)
    pr = i * _BP + lax.broadcasted_iota(jnp.int32, (_BP, 1), 0)
    gid = lax.broadcasted_iota(jnp.int32, (_BP, N_GRAPHS), 1)
    bat = bat_ref[...]
    sel_e = jnp.where((bat[:, 0:1] == gid) & (2 * pr < N_NODES), 1.0, 0.0)
    sel_o = jnp.where((bat[:, 1:2] == gid) & (2 * pr + 1 < N_NODES), 1.0, 0.0)
    pooled[...] += (
        lax.dot_general(sel_e, h2[:, :HIDDEN], (((0,), (0,)), ((), ())),
                        preferred_element_type=jnp.float32)
        + lax.dot_general(sel_o, h2[:, HIDDEN:], (((0,), (0,)), ((), ())),
                          preferred_element_type=jnp.float32))

    @pl.when(i == _GRID - 1)
    def _():
        o_ref[...] = lax.dot_general(
            pooled[...], w3_ref[...], (((1,), (1,)), ((), ())),
            preferred_element_type=jnp.float32) + b3_ref[...]


_pair_spec = pl.BlockSpec((_BP, 128), lambda i: (i, 0))
_d2_spec = pl.BlockSpec((_BP, 2), lambda i: (i, 0))
_w_spec = pl.BlockSpec((128, 128), lambda i: (0, 0))
_b_spec = pl.BlockSpec((1, 128), lambda i: (0, 0))
_pair_shape = jax.ShapeDtypeStruct((NPAIR, 128), jnp.float32)


def _run_tc_k1(h0p, dp0, dp1, Wbd):
    return pl.pallas_call(
        _tc_k1,
        grid=(_GRID,),
        in_specs=[_pair_spec, _d2_spec, _d2_spec, _w_spec],
        out_specs=_pair_spec,
        out_shape=_pair_shape,
    )(h0p, dp0, dp1, Wbd)


def _run_tc_k2(accp, hwp, dp0, dp1, Wbd, bp):
    return pl.pallas_call(
        _tc_k2,
        grid=(_GRID,),
        in_specs=[_pair_spec, _pair_spec, _d2_spec, _d2_spec, _w_spec,
                  _b_spec],
        out_specs=_pair_spec,
        out_shape=_pair_shape,
    )(accp, hwp, dp0, dp1, Wbd, bp)


def _run_tc_k3(accp, hwp, dp0, dp1, bp, batp, W3, b3):
    return pl.pallas_call(
        _tc_k3,
        grid=(_GRID,),
        in_specs=[_pair_spec, _pair_spec, _d2_spec, _d2_spec, _b_spec,
                  _d2_spec,
                  pl.BlockSpec((OUT_DIM, HIDDEN), lambda i: (0, 0)),
                  pl.BlockSpec((1, OUT_DIM), lambda i: (0, 0))],
        out_specs=pl.BlockSpec((N_GRAPHS, OUT_DIM), lambda i: (0, 0)),
        out_shape=jax.ShapeDtypeStruct((N_GRAPHS, OUT_DIM), jnp.float32),
        scratch_shapes=[pltpu.VMEM((N_GRAPHS, HIDDEN), jnp.float32)],
    )(accp, hwp, dp0, dp1, bp, batp, W3, b3)


def _blockdiag2(W):
    z = jnp.zeros((HIDDEN, HIDDEN), jnp.float32)
    return jnp.concatenate(
        [jnp.concatenate([W, z], axis=1),
         jnp.concatenate([z, W], axis=1)], axis=0)


def kernel(x, edge_index, batch, table, W1, b1, W2, b2, W3, b3):
    x = x.astype(jnp.int32)
    src = edge_index[0].astype(jnp.int32)
    dst = edge_index[1].astype(jnp.int32)
    batch = batch.astype(jnp.int32)

    npx = X_PAD - N_NODES * TOK_PER_NODE
    pad_x = (jnp.arange(npx, dtype=jnp.int32) * 9973) % VOCAB
    xp = jnp.concatenate([x.reshape(-1), pad_x]).reshape(X_PAD // 128, 128)

    eip = jnp.concatenate(
        [jnp.stack([src, dst]), jnp.zeros((2, 1024), jnp.int32)], axis=1)

    batp = jnp.concatenate(
        [batch, jnp.zeros((NACC - N_NODES,), jnp.int32)]).reshape(NPAIR, 2)

    h0, deg0, deg1 = _sc_emb_deg(table, xp, eip)
    h0p = h0.reshape(NPAIR, 128)
    dp0 = deg0.reshape(NPAIR, 2)
    dp1 = deg1.reshape(NPAIR, 2)

    Wbd1 = _blockdiag2(W1)
    Wbd2 = _blockdiag2(W2)
    bp1 = jnp.concatenate([b1, b1]).reshape(1, 128)
    bp2 = jnp.concatenate([b2, b2]).reshape(1, 128)

    hwp1 = _run_tc_k1(h0p, dp0, dp1, Wbd1)
    acc = _sc_edge(hwp1.reshape(2 * NACC, 32), eip)
    hwp2 = _run_tc_k2(acc.reshape(NPAIR, 128), hwp1, dp0, dp1, Wbd2, bp1)
    acc = _sc_edge(hwp2.reshape(2 * NACC, 32), eip)
    out = _run_tc_k3(acc.reshape(NPAIR, 128), hwp2, dp0, dp1, bp2, batp, W3,
                     b3.reshape(1, OUT_DIM))
    return out


# speedup vs baseline: 1.2743x; 1.2743x over previous
import functools

import jax
import jax.numpy as jnp
from jax import lax
from jax.experimental import pallas as pl
from jax.experimental.pallas import tpu as pltpu
from jax.experimental.pallas import tpu_sc as plsc

N_NODES = 50000
N_EDGES = 800000
N_GRAPHS = 512
VOCAB = 100000
EMB_DIM = 16
TOK_PER_NODE = 4
HIDDEN = 64
OUT_DIM = 1000

NACC = 50176
SLAB = NACC // 16
E_PAD = 819200
EW_PER_TILE = 400
X_PAD = 200704
DW_PER_TILE = 200
NPAIR = NACC // 2

_mesh = plsc.VectorSubcoreMesh(core_axis_name="c", subcore_axis_name="s")
_sc_params = pltpu.CompilerParams(use_tc_tiling_on_sc=False)


@functools.partial(
    pl.kernel,
    mesh=_mesh,
    compiler_params=_sc_params,
    out_type=[
        jax.ShapeDtypeStruct((X_PAD, EMB_DIM), jnp.float32),
        jax.ShapeDtypeStruct((NACC,), jnp.float32),
        jax.ShapeDtypeStruct((NACC,), jnp.float32),
    ],
    scratch_types=[
        pltpu.VMEM((896, EMB_DIM), jnp.float32),
        pltpu.VMEM((7, 128), jnp.int32),
        pltpu.VMEM((8, 128), jnp.int32),
        pltpu.VMEM((128,), jnp.float32),
        pltpu.VMEM((SLAB,), jnp.float32),
        pltpu.VMEM_SHARED((NACC,), jnp.float32),
        pltpu.SemaphoreType.DMA,
    ],
)
def _sc_emb_deg(table, xp, dstp, h0, deg0, deg1, rows, xbuf,
                dstbuf, ones, zbuf, deg_sh, gsem):
    c = lax.axis_index("c")
    s = lax.axis_index("s")
    w = s * 2 + c

    def zf(i, _):
        zbuf[pl.ds(i * 16, 16)] = jnp.zeros((16,), jnp.float32)
        return 0
    lax.fori_loop(0, SLAB // 16, zf, 0)
    for i in range(8):
        ones[pl.ds(i * 16, 16)] = jnp.ones((16,), jnp.float32)
    pltpu.sync_copy(zbuf, deg_sh.at[pl.ds(s * SLAB, SLAB)])
    plsc.subcore_barrier()

    def emb(g, _):
        pltpu.sync_copy(xp.at[pl.ds(w * 49 + g * 7, 7)], xbuf)
        cps = []
        for j in range(7):
            cps.append(pltpu.async_copy(
                table.at[xbuf.at[j]], rows.at[pl.ds(j * 128, 128)], gsem))
        for cp in cps:
            cp.wait()
        pltpu.sync_copy(rows, h0.at[pl.ds(w * 6272 + g * 896, 896)])
        return 0
    lax.fori_loop(0, 7, emb, 0)

    def deg(g, _):
        pltpu.sync_copy(dstp.at[pl.ds(w * DW_PER_TILE + g * 8, 8)], dstbuf)
        for j in range(8):
            pltpu.sync_copy(ones, deg_sh.at[dstbuf.at[j]], add=True)
        return 0
    lax.fori_loop(0, DW_PER_TILE // 8, deg, 0)

    plsc.subcore_barrier()

    @pl.when(c == 0)
    def _():
        pltpu.sync_copy(deg_sh.at[pl.ds(s * SLAB, SLAB)],
                        deg0.at[pl.ds(s * SLAB, SLAB)])

    @pl.when(c == 1)
    def _():
        pltpu.sync_copy(deg_sh.at[pl.ds(s * SLAB, SLAB)],
                        deg1.at[pl.ds(s * SLAB, SLAB)])


_NG = EW_PER_TILE // 2
_NCHUNK = EW_PER_TILE // 8


@functools.partial(
    pl.kernel,
    mesh=_mesh,
    compiler_params=_sc_params,
    out_type=jax.ShapeDtypeStruct((NACC, 2, 32), jnp.float32),
    scratch_types=[
        pltpu.VMEM((16, 128), jnp.int32),
        pltpu.VMEM((16, 128), jnp.int32),
        pltpu.VMEM((4, 128), jnp.int32),
        pltpu.VMEM((4, 128, 32), jnp.float32),
        pltpu.VMEM((196, 32), jnp.float32),
        pltpu.VMEM_SHARED((NACC, 32), jnp.float32),
        pltpu.SemaphoreType.DMA,
        pltpu.SemaphoreType.DMA,
        pltpu.SemaphoreType.DMA,
    ],
)
def _sc_edge(hh, srcp, dstp, acc3, srcb, dstb, idxg, rows, zb, acc_sh,
             gsem, ssem, isem):
    c = lax.axis_index("c")
    s = lax.axis_index("s")
    base = s * EW_PER_TILE

    def zf(r, _):
        zb[r, pl.ds(0, 16)] = jnp.zeros((16,), jnp.float32)
        zb[r, pl.ds(16, 16)] = jnp.zeros((16,), jnp.float32)
        return 0
    lax.fori_loop(0, 196, zf, 0)
    for k in range(16):
        pltpu.sync_copy(zb, acc_sh.at[pl.ds(s * SLAB + k * 196, 196)])
    plsc.subcore_barrier()

    pltpu.sync_copy(srcp.at[pl.ds(base, 8)], srcb.at[pl.ds(0, 8)])
    pltpu.sync_copy(dstp.at[pl.ds(base, 8)], dstb.at[pl.ds(0, 8)])
    pltpu.async_copy(srcp.at[pl.ds(base + 8, 8)], srcb.at[pl.ds(8, 8)], isem)
    pltpu.async_copy(dstp.at[pl.ds(base + 8, 8)], dstb.at[pl.ds(8, 8)], isem)
    for j in range(2):
        for v in range(8):
            sl = pl.ds(v * 16, 16)
            idxg[j, sl] = srcb[j, sl] * 2 + c
    for j in range(2):
        pltpu.async_copy(hh.at[idxg.at[j]], rows.at[j], gsem)

    def step(g, _):
        slot = lax.rem(g, 2)
        sb = slot * 2
        nb = 2 - sb
        pos = lax.rem(g, 4)
        k = g // 4
        kslot = lax.rem(k, 2)

        for j in range(2):
            pltpu.make_async_copy(
                hh.at[idxg.at[sb + j]], rows.at[sb + j], gsem).wait()

        @pl.when((pos == 3) & (g < _NG - 1))
        def _():
            pltpu.make_async_copy(
                srcp.at[pl.ds(base, 8)], srcb.at[pl.ds(0, 8)], isem).wait()
            pltpu.make_async_copy(
                dstp.at[pl.ds(base, 8)], dstb.at[pl.ds(0, 8)], isem).wait()

        @pl.when(g < _NG - 1)
        def _():
            for j in range(2):
                w = (g + 1) * 2 + j
                r = lax.rem(w, 16)
                for v in range(8):
                    sl = pl.ds(v * 16, 16)
                    idxg[nb + j, sl] = srcb[r, sl] * 2 + c
            for j in range(2):
                pltpu.async_copy(hh.at[idxg.at[nb + j]], rows.at[nb + j],
                                 gsem)

        scs = []
        for j in range(2):
            w = g * 2 + j
            scs.append(pltpu.async_copy(
                rows.at[sb + j], acc_sh.at[dstb.at[lax.rem(w, 16)]], ssem,
                add=True))
        for cp in scs:
            cp.wait()

        @pl.when((pos == 3) & (k + 2 < _NCHUNK))
        def _():
            pltpu.async_copy(srcp.at[pl.ds(base + (k + 2) * 8, 8)],
                             srcb.at[pl.ds(kslot * 8, 8)], isem)
            pltpu.async_copy(dstp.at[pl.ds(base + (k + 2) * 8, 8)],
                             dstb.at[pl.ds(kslot * 8, 8)], isem)
        return 0
    lax.fori_loop(0, _NG, step, 0)

    plsc.subcore_barrier()
    pltpu.sync_copy(acc_sh.at[pl.ds(s * SLAB, SLAB)],
                    acc3.at[pl.ds(s * SLAB, SLAB), c])


_BP = 512
_GRID = NPAIR // _BP


def _dinv_pair(d_ref, e_ref):
    d2 = lax.rsqrt(d_ref[...] + e_ref[...] + 1.0)
    return jnp.concatenate(
        [jnp.broadcast_to(d2[:, 0:1], (_BP, HIDDEN)),
         jnp.broadcast_to(d2[:, 1:2], (_BP, HIDDEN))], axis=1)


def _tc_k1(h0_ref, d_ref, e_ref, w_ref, o_ref):
    hw = jnp.dot(h0_ref[...], w_ref[...], preferred_element_type=jnp.float32)
    o_ref[...] = hw * _dinv_pair(d_ref, e_ref)


def _tc_k2(acc_ref, hwp_ref, d_ref, e_ref, w_ref, b_ref, o_ref):
    dinv = _dinv_pair(d_ref, e_ref)
    h1 = jnp.maximum((acc_ref[...] + hwp_ref[...]) * dinv + b_ref[...], 0.0)
    hw = jnp.dot(h1, w_ref[...], preferred_element_type=jnp.float32)
    o_ref[...] = hw * dinv


def _tc_k3(acc_ref, hwp_ref, d_ref, e_ref, b_ref, bat_ref, w3_ref,
           b3_ref, o_ref, pooled):
    i = pl.program_id(0)

    @pl.when(i == 0)
    def _():
        pooled[...] = jnp.zeros((N_GRAPHS, HIDDEN), jnp.float32)

    dinv = _dinv_pair(d_ref, e_ref)
    h2 = jnp.maximum((acc_ref[...] + hwp_ref[...]) * dinv + b_ref[...], 0.0)
    pr = i * _BP + lax.broadcasted_iota(jnp.int32, (_BP, 1), 0)
    gid = lax.broadcasted_iota(jnp.int32, (_BP, N_GRAPHS), 1)
    bat = bat_ref[...]
    sel_e = jnp.where((bat[:, 0:1] == gid) & (2 * pr < N_NODES), 1.0, 0.0)
    sel_o = jnp.where((bat[:, 1:2] == gid) & (2 * pr + 1 < N_NODES), 1.0, 0.0)
    pooled[...] += (
        lax.dot_general(sel_e, h2[:, :HIDDEN], (((0,), (0,)), ((), ())),
                        preferred_element_type=jnp.float32)
        + lax.dot_general(sel_o, h2[:, HIDDEN:], (((0,), (0,)), ((), ())),
                          preferred_element_type=jnp.float32))

    @pl.when(i == _GRID - 1)
    def _():
        o_ref[...] = lax.dot_general(
            pooled[...], w3_ref[...], (((1,), (1,)), ((), ())),
            preferred_element_type=jnp.float32) + b3_ref[...]


_pair_spec = pl.BlockSpec((_BP, 128), lambda i: (i, 0))
_d2_spec = pl.BlockSpec((_BP, 2), lambda i: (i, 0))
_w_spec = pl.BlockSpec((128, 128), lambda i: (0, 0))
_b_spec = pl.BlockSpec((1, 128), lambda i: (0, 0))
_pair_shape = jax.ShapeDtypeStruct((NPAIR, 128), jnp.float32)


def _run_tc_k1(h0p, dp0, dp1, Wbd):
    return pl.pallas_call(
        _tc_k1,
        grid=(_GRID,),
        in_specs=[_pair_spec, _d2_spec, _d2_spec, _w_spec],
        out_specs=_pair_spec,
        out_shape=_pair_shape,
    )(h0p, dp0, dp1, Wbd)


def _run_tc_k2(accp, hwp, dp0, dp1, Wbd, bp):
    return pl.pallas_call(
        _tc_k2,
        grid=(_GRID,),
        in_specs=[_pair_spec, _pair_spec, _d2_spec, _d2_spec, _w_spec,
                  _b_spec],
        out_specs=_pair_spec,
        out_shape=_pair_shape,
    )(accp, hwp, dp0, dp1, Wbd, bp)


def _run_tc_k3(accp, hwp, dp0, dp1, bp, batp, W3, b3):
    return pl.pallas_call(
        _tc_k3,
        grid=(_GRID,),
        in_specs=[_pair_spec, _pair_spec, _d2_spec, _d2_spec, _b_spec,
                  _d2_spec,
                  pl.BlockSpec((OUT_DIM, HIDDEN), lambda i: (0, 0)),
                  pl.BlockSpec((1, OUT_DIM), lambda i: (0, 0))],
        out_specs=pl.BlockSpec((N_GRAPHS, OUT_DIM), lambda i: (0, 0)),
        out_shape=jax.ShapeDtypeStruct((N_GRAPHS, OUT_DIM), jnp.float32),
        scratch_shapes=[pltpu.VMEM((N_GRAPHS, HIDDEN), jnp.float32)],
    )(accp, hwp, dp0, dp1, bp, batp, W3, b3)


def _blockdiag2(W):
    z = jnp.zeros((HIDDEN, HIDDEN), jnp.float32)
    return jnp.concatenate(
        [jnp.concatenate([W, z], axis=1),
         jnp.concatenate([z, W], axis=1)], axis=0)


def kernel(x, edge_index, batch, table, W1, b1, W2, b2, W3, b3):
    x = x.astype(jnp.int32)
    src = edge_index[0].astype(jnp.int32)
    dst = edge_index[1].astype(jnp.int32)
    batch = batch.astype(jnp.int32)

    npx = X_PAD - N_NODES * TOK_PER_NODE
    pad_x = (jnp.arange(npx, dtype=jnp.int32) * 9973) % VOCAB
    xp = jnp.concatenate([x.reshape(-1), pad_x]).reshape(X_PAD // 128, 128)

    ne_pad = E_PAD - N_EDGES
    pad_src = (jnp.arange(ne_pad, dtype=jnp.int32) * 37) % N_NODES
    pad_dst = N_NODES + (jnp.arange(ne_pad, dtype=jnp.int32) % (NACC - N_NODES))
    srcp = jnp.concatenate([src, pad_src]).reshape(E_PAD // 128, 128)
    dstp = jnp.concatenate([dst, pad_dst]).reshape(E_PAD // 128, 128)

    batp = jnp.concatenate(
        [batch, jnp.zeros((NACC - N_NODES,), jnp.int32)]).reshape(NPAIR, 2)

    h0, deg0, deg1 = _sc_emb_deg(table, xp, dstp)
    h0p = h0.reshape(NPAIR, 128)
    dp0 = deg0.reshape(NPAIR, 2)
    dp1 = deg1.reshape(NPAIR, 2)

    Wbd1 = _blockdiag2(W1)
    Wbd2 = _blockdiag2(W2)
    bp1 = jnp.concatenate([b1, b1]).reshape(1, 128)
    bp2 = jnp.concatenate([b2, b2]).reshape(1, 128)

    hwp1 = _run_tc_k1(h0p, dp0, dp1, Wbd1)
    acc = _sc_edge(hwp1.reshape(2 * NACC, 32), srcp, dstp)
    hwp2 = _run_tc_k2(acc.reshape(NPAIR, 128), hwp1, dp0, dp1, Wbd2, bp1)
    acc = _sc_edge(hwp2.reshape(2 * NACC, 32), srcp, dstp)
    out = _run_tc_k3(acc.reshape(NPAIR, 128), hwp2, dp0, dp1, bp2, batp, W3,
                     b3.reshape(1, OUT_DIM))
    return out
